# baseline (speedup 1.0000x reference)
import jax
import jax.numpy as jnp
from jax import lax
from jax.experimental import pallas as pl
from jax.experimental.pallas import tpu as pltpu

B = 1024
N_CTX = 4
D = 512
ROWS = 77
P_PRE, P_MID, P_SUF = 5, 2, 62
OFF_CLS = P_PRE
OFF_MID = OFF_CLS + N_CTX
OFF_CLO = OFF_MID + P_MID
OFF_SUF = OFF_CLO + N_CTX

EPB = 16
STEPS = B // EPB
RING = 6


def _asm_body(lbl_s, clo_s, cls_hbm, clo_hbm, tmpl_ref, out_hbm,
              stage, cls_v, clo_v, out_sem, g_sem):
    i = pl.program_id(0)
    b0 = i * EPB
    cur = lax.rem(i, RING)
    base = cur * EPB

    copies = []
    for e in range(EPB):
        c1 = pltpu.make_async_copy(cls_hbm.at[lbl_s[b0 + e]], cls_v.at[e],
                                   g_sem.at[0, e])
        c2 = pltpu.make_async_copy(clo_hbm.at[clo_s[b0 + e]], clo_v.at[e],
                                   g_sem.at[1, e])
        c1.start()
        c2.start()
        copies.append((c1, c2))

    @pl.when(i < RING)
    def _prefill():
        for e in range(EPB):
            stage[pl.ds(base + e, 1)] = tmpl_ref[...].reshape(1, ROWS, D)

    @pl.when(i >= RING)
    def _recycle():
        pltpu.make_async_copy(stage.at[pl.ds(base, EPB)],
                              out_hbm.at[pl.ds(0, EPB)],
                              out_sem.at[cur]).wait()

    for e in range(EPB):
        c1, c2 = copies[e]
        c1.wait()
        c2.wait()
        stage[pl.ds(base + e, 1), OFF_CLS:OFF_CLS + N_CTX] = (
            cls_v[e].reshape(1, N_CTX, D))
        stage[pl.ds(base + e, 1), OFF_CLO:OFF_CLO + N_CTX] = (
            clo_v[e].reshape(1, N_CTX, D))

    pltpu.make_async_copy(stage.at[pl.ds(base, EPB)],
                          out_hbm.at[pl.ds(b0, EPB)],
                          out_sem.at[cur]).start()

    @pl.when(i == STEPS - 1)
    def _drain():
        for r in range(RING):
            pltpu.make_async_copy(stage.at[pl.ds(r * EPB, EPB)],
                                  out_hbm.at[pl.ds(0, EPB)],
                                  out_sem.at[r]).wait()


@jax.jit
def _prompt_assemble(label, cloth_label, cls_ctx, clo_ctx, tmpl_full):
    grid_spec = pltpu.PrefetchScalarGridSpec(
        num_scalar_prefetch=2,
        grid=(STEPS,),
        in_specs=[
            pl.BlockSpec(memory_space=pltpu.MemorySpace.HBM),
            pl.BlockSpec(memory_space=pltpu.MemorySpace.HBM),
            pl.BlockSpec((ROWS, D), lambda i, lbl, clo: (0, 0)),
        ],
        out_specs=pl.BlockSpec(memory_space=pltpu.MemorySpace.HBM),
        scratch_shapes=[
            pltpu.VMEM((RING * EPB, ROWS, D), jnp.float32),
            pltpu.VMEM((EPB, N_CTX, D), jnp.float32),
            pltpu.VMEM((EPB, N_CTX, D), jnp.float32),
            pltpu.SemaphoreType.DMA((RING,)),
            pltpu.SemaphoreType.DMA((2, EPB)),
        ],
    )
    return pl.pallas_call(
        _asm_body,
        grid_spec=grid_spec,
        out_shape=jax.ShapeDtypeStruct((B, ROWS, D), jnp.float32),
        compiler_params=pltpu.CompilerParams(
            dimension_semantics=("arbitrary",)),
    )(label, cloth_label, cls_ctx, clo_ctx, tmpl_full)


def kernel(label, cloth_label, cls_ctx, cls_cloth_ctx,
           token_prefix, token_middle, token_suffix):
    zeros4 = jnp.zeros((N_CTX, D), jnp.float32)
    tmpl_full = jnp.concatenate(
        [token_prefix.reshape(P_PRE, D), zeros4,
         token_middle.reshape(P_MID, D), zeros4,
         token_suffix.reshape(P_SUF, D)], axis=0)
    out = _prompt_assemble(label.astype(jnp.int32),
                           cloth_label.astype(jnp.int32),
                           cls_ctx, cls_cloth_ctx, tmpl_full)
    return (out, 17)
